# baseline (speedup 1.0000x reference)
import jax
import jax.numpy as jnp
from jax import lax
from jax.experimental import pallas as pl
from jax.experimental.pallas import tpu as pltpu
from jax.experimental.pallas import tpu_sc as plsc

NC = 2
NS = 16
NW = NC * NS
NBUF = 3
LANES = 16
IXA = 96
IXB = 104
IBLK = 8
SUPER = 24


def _make_body(chunks, seq, d):
    nvec = d // LANES
    nblk = chunks // IBLK

    def body(x_hbm, w_hbm, pos_hbm, out_hbm, pos_v,
             buf0, buf1, buf2, ib0, ib1, ib2, lb0, lb1,
             gsem0, gsem1, gsem2, wsem0, wsem1, wsem2,
             isem0, isem1, isem2):
        bufs = (buf0, buf1, buf2)
        ibufs = (ib0, ib1, ib2)
        lbufs = (lb0, lb1)
        gsems = (gsem0, gsem1, gsem2)
        wsems = (wsem0, wsem1, wsem2)
        isems = (isem0, isem1, isem2)

        wid = lax.axis_index("s") * NC + lax.axis_index("c")
        pltpu.sync_copy(pos_hbm, pos_v)

        def idx_start(r, j):
            row = pl.multiple_of(chunks * wid + IBLK * r, IBLK)
            pltpu.make_async_copy(x_hbm.at[pl.ds(row, IBLK)],
                                  ibufs[j], isems[j]).start()

        def idx_wait(j):
            pltpu.make_async_copy(x_hbm.at[pl.ds(0, IBLK)],
                                  ibufs[j], isems[j]).wait()

        def detile_row(j, row, l):
            dst = lbufs[l]
            src = ibufs[j]
            for c0 in range(0, seq - LANES + 1, LANES):
                dst[pl.ds(c0, LANES)] = src[row, pl.ds(c0, LANES)]
            if seq % LANES:
                c0 = seq - LANES
                dst[pl.ds(c0, LANES)] = src[row, pl.ds(c0, LANES)]

        def gather_descs(b, l):
            return (
                pltpu.make_async_copy(w_hbm.at[lbufs[l].at[pl.ds(0, IXA)]],
                                      bufs[b].at[pl.ds(0, IXA)], gsems[b]),
                pltpu.make_async_copy(w_hbm.at[lbufs[l].at[pl.ds(IXA, IXB)]],
                                      bufs[b].at[pl.ds(IXA, IXB)], gsems[b]),
            )

        def gather_start(b, j, row, l):
            detile_row(j, row, l)
            for dsc in gather_descs(b, l):
                dsc.start()

        def gather_wait(b):
            for dsc in gather_descs(b, 0):
                dsc.wait()

        def wb_start(k, b):
            pltpu.make_async_copy(bufs[b], out_hbm.at[wid, k], wsems[b]).start()

        def wb_wait(b):
            pltpu.make_async_copy(bufs[b], out_hbm.at[wid, 0], wsems[b]).wait()

        def add_pos(b):
            rows = bufs[b]

            @pl.loop(0, seq, unroll=2)
            def _(i):
                for j in range(nvec):
                    sl = pl.ds(j * LANES, LANES)
                    plsc.addupdate(rows.at[i, sl], pos_v[i, sl])

        def step(k, c, dyn):
            b = c % NBUF
            bn = (c + 1) % NBUF

            if dyn:
                @pl.when(k >= 2)
                def _():
                    wb_wait(bn)
            elif k >= 2:
                wb_wait(bn)

            if not dyn and k + 1 >= chunks:
                pass
            else:
                if (c + 1) % IBLK == 0:
                    idx_wait((((c + 1) // IBLK) % NBUF))
                gather_start(bn, ((c + 1) // IBLK) % NBUF, (c + 1) % IBLK,
                             (c + 1) % 2)

            gather_wait(b)

            if (c + 1) % IBLK == 0:
                j2 = (c // IBLK + 2) % NBUF
                if dyn:
                    @pl.when(k < (nblk - 2) * IBLK)
                    def _():
                        idx_start(k // IBLK + 2, j2)
                elif k // IBLK + 2 < nblk:
                    idx_start(k // IBLK + 2, j2)

            add_pos(b)
            wb_start(k, b)

        idx_start(0, 0)
        idx_start(1, 1)
        idx_wait(0)
        gather_start(0, 0, 0, 0)

        main = (chunks // SUPER) * SUPER
        if main == chunks:
            main -= SUPER

        @pl.loop(0, main, step=SUPER)
        def _(g):
            for c in range(SUPER):
                step(g + c, c, dyn=True)

        for k in range(main, chunks):
            step(k, k % SUPER, dyn=False)

        for k in range(chunks - 2, chunks):
            wb_wait(k % NBUF)

    return body


def kernel(x, W, pos):
    B, S = x.shape
    V, d = W.shape
    n = B * S
    per_w = n // NW
    chunks = per_w // S
    assert n == NW * chunks * S and S == IXA + IXB and d % LANES == 0
    assert chunks % IBLK == 0 and SUPER % NBUF == 0 and SUPER % IBLK == 0

    x_r = x.astype(jnp.int32)
    mesh = plsc.VectorSubcoreMesh(
        core_axis_name="c", subcore_axis_name="s",
        num_cores=NC, num_subcores=NS)
    run = pl.kernel(
        _make_body(chunks, S, d),
        out_type=jax.ShapeDtypeStruct((NW, chunks, S, d), jnp.float32),
        mesh=mesh,
        scratch_types=[
            pltpu.VMEM((S, d), jnp.float32),
        ] + [pltpu.VMEM((S, d), jnp.float32)] * NBUF
          + [pltpu.VMEM((IBLK, S), jnp.int32)] * NBUF
          + [pltpu.VMEM((S,), jnp.int32)] * 2
          + [pltpu.SemaphoreType.DMA] * (3 * NBUF),
    )
    out = run(x_r, W, pos)
    return out.reshape(B, S, d)


# speedup vs baseline: 1.0034x; 1.0034x over previous
import jax
import jax.numpy as jnp
from jax import lax
from jax.experimental import pallas as pl
from jax.experimental.pallas import tpu as pltpu
from jax.experimental.pallas import tpu_sc as plsc

NC = 2
NS = 16
NW = NC * NS
NBUF = 3
LANES = 16
IXA = 96
IXB = 104
IBLK = 8
SUPER = 24


def _make_body(chunks, seq, d):
    nvec = d // LANES
    nblk = chunks // IBLK

    def body(x_hbm, w_hbm, pos_hbm, out_hbm, pos_v,
             buf0, buf1, buf2, ib0, ib1, ib2, lb0, lb1,
             gsem0, gsem1, gsem2, wsem0, wsem1, wsem2,
             isem0, isem1, isem2):
        bufs = (buf0, buf1, buf2)
        ibufs = (ib0, ib1, ib2)
        lbufs = (lb0, lb1)
        gsems = (gsem0, gsem1, gsem2)
        wsems = (wsem0, wsem1, wsem2)
        isems = (isem0, isem1, isem2)

        wid = lax.axis_index("s") * NC + lax.axis_index("c")
        pltpu.sync_copy(pos_hbm, pos_v)

        def idx_start(r, j):
            row = pl.multiple_of(chunks * wid + IBLK * r, IBLK)
            pltpu.make_async_copy(x_hbm.at[pl.ds(row, IBLK)],
                                  ibufs[j], isems[j]).start()

        def idx_wait(j):
            pltpu.make_async_copy(x_hbm.at[pl.ds(0, IBLK)],
                                  ibufs[j], isems[j]).wait()

        def detile_row(j, row, l):
            dst = lbufs[l]
            src = ibufs[j]
            for c0 in range(0, seq - LANES + 1, LANES):
                dst[pl.ds(c0, LANES)] = src[row, pl.ds(c0, LANES)]
            if seq % LANES:
                c0 = seq - LANES
                dst[pl.ds(c0, LANES)] = src[row, pl.ds(c0, LANES)]

        def gather_descs(b, l):
            return (
                pltpu.make_async_copy(w_hbm.at[lbufs[l].at[pl.ds(0, IXA)]],
                                      bufs[b].at[pl.ds(0, IXA)], gsems[b]),
                pltpu.make_async_copy(w_hbm.at[lbufs[l].at[pl.ds(IXA, IXB)]],
                                      bufs[b].at[pl.ds(IXA, IXB)], gsems[b]),
            )

        def gather_start(b, j, row, l):
            detile_row(j, row, l)
            for dsc in gather_descs(b, l):
                dsc.start()

        def gather_wait(b):
            for dsc in gather_descs(b, 0):
                dsc.wait()

        def wb_start(k, b):
            pltpu.make_async_copy(bufs[b], out_hbm.at[wid, k], wsems[b]).start()

        def wb_wait(b):
            pltpu.make_async_copy(bufs[b], out_hbm.at[wid, 0], wsems[b]).wait()

        def add_pos(b):
            rows = bufs[b]

            @pl.loop(0, seq, unroll=2)
            def _(i):
                for j in range(nvec):
                    sl = pl.ds(j * LANES, LANES)
                    plsc.addupdate(rows.at[i, sl], pos_v[i, sl])

        def step(k, c, dyn):
            b = c % NBUF
            bn = (c + 1) % NBUF

            if dyn:
                @pl.when(k >= 2)
                def _():
                    wb_wait(bn)
            elif k >= 2:
                wb_wait(bn)

            if not dyn and k + 1 >= chunks:
                pass
            else:
                if (c + 1) % IBLK == 0:
                    idx_wait((((c + 1) // IBLK) % NBUF))
                gather_start(bn, ((c + 1) // IBLK) % NBUF, (c + 1) % IBLK,
                             (c + 1) % 2)

            gather_wait(b)

            if (c + 1) % IBLK == 0:
                j2 = (c // IBLK + 2) % NBUF
                if dyn:
                    @pl.when(k < (nblk - 2) * IBLK)
                    def _():
                        idx_start(k // IBLK + 2, j2)
                elif k // IBLK + 2 < nblk:
                    idx_start(k // IBLK + 2, j2)

            add_pos(b)
            wb_start(k, b)

        idx_start(0, 0)
        idx_start(1, 1)
        idx_wait(0)
        gather_start(0, 0, 0, 0)

        main = (chunks // SUPER) * SUPER
        if main == chunks:
            main -= SUPER

        @pl.loop(0, main, step=SUPER)
        def _(g):
            for c in range(SUPER):
                step(g + c, c, dyn=True)

        for k in range(main, chunks):
            step(k, k % SUPER, dyn=False)

        for k in range(chunks - 2, chunks):
            wb_wait(k % NBUF)

    return body


def kernel(x, W, pos):
    B, S = x.shape
    V, d = W.shape
    n = B * S
    per_w = n // NW
    chunks = per_w // S
    assert n == NW * chunks * S and S == IXA + IXB and d % LANES == 0
    assert chunks % IBLK == 0 and SUPER % NBUF == 0 and SUPER % IBLK == 0

    x_r = x.astype(jnp.int32)
    mesh = plsc.VectorSubcoreMesh(
        core_axis_name="c", subcore_axis_name="s",
        num_cores=NC, num_subcores=NS)
    run = pl.kernel(
        _make_body(chunks, S, d),
        out_type=jax.ShapeDtypeStruct((NW, chunks, S, d), jnp.float32),
        mesh=mesh,
        compiler_params=pltpu.CompilerParams(use_tc_tiling_on_sc=True),
        scratch_types=[
            pltpu.VMEM((S, d), jnp.float32),
        ] + [pltpu.VMEM((S, d), jnp.float32)] * NBUF
          + [pltpu.VMEM((IBLK, S), jnp.int32)] * NBUF
          + [pltpu.VMEM((S,), jnp.int32)] * 2
          + [pltpu.SemaphoreType.DMA] * (3 * NBUF),
    )
    out = run(x_r, W, pos)
    return out.reshape(B, S, d)
